# baseline (speedup 1.0000x reference)
import numpy as np
import jax
import jax.numpy as jnp
from jax import lax
from jax.experimental import pallas as pl
from jax.experimental.pallas import tpu as pltpu

DIM = 256
HEADS = 8
HD = DIM // HEADS
WIN = 8
E = 8
SCALE = HD ** -0.5
RHID = 128
BLK = 512
NWIN = 8
N = WIN * WIN


def _rope_tables():
    d = HD // 2
    half = d // 2
    inv = 1.0 / (10000.0 ** (np.arange(half, dtype=np.float64) / half))
    hpos = np.repeat(np.arange(WIN), WIN).astype(np.float64)
    wpos = np.tile(np.arange(WIN), WIN).astype(np.float64)
    ah = hpos[:, None] * inv[None, :]
    aw = wpos[:, None] * inv[None, :]
    cos = np.concatenate([np.cos(ah), np.cos(ah), np.cos(aw), np.cos(aw)], axis=-1)
    sin = np.concatenate([np.sin(ah), np.sin(ah), np.sin(aw), np.sin(aw)], axis=-1)
    t = np.arange(BLK)
    p = (t // 64) * WIN + (t % WIN)
    cos_b = np.tile(cos[p], (1, 2 * HEADS)).astype(np.float32)
    sin_b = np.tile(sin[p], (1, 2 * HEADS)).astype(np.float32)
    return cos_b, sin_b


_COS_B, _SIN_B = _rope_tables()


def _rot_half_qk(x):
    pieces = []
    for g in range(2 * HEADS):
        b = g * HD
        pieces += [-x[:, b + 8:b + 16], x[:, b:b + 8],
                   -x[:, b + 24:b + 32], x[:, b + 16:b + 24]]
    return jnp.concatenate(pieces, axis=1)


def _body(x_ref, wqkv_ref, bqkv_ref, wproj_ref, bproj_ref,
          wr1_ref, br1_ref, wr2_ref, br2_ref, cos_ref, sin_ref,
          out_ref, qkv_s, att_s):
    x = x_ref[...]

    hid = jnp.maximum(
        jnp.dot(x, wr1_ref[...], preferred_element_type=jnp.float32)
        + br1_ref[...], 0.0)
    logits = (jnp.dot(hid, wr2_ref[...], preferred_element_type=jnp.float32)
              + br2_ref[...])
    mx = jnp.max(logits, axis=-1, keepdims=True)
    ex = jnp.exp(logits - mx)
    probs = ex / jnp.sum(ex, axis=-1, keepdims=True)
    pmax = jnp.max(probs, axis=-1, keepdims=True)
    masks = []
    found = jnp.zeros((BLK, 1), jnp.float32)
    for e in range(E):
        col = probs[:, e:e + 1]
        is_max = jnp.where(col >= pmax, 1.0, 0.0) * (1.0 - found)
        masks.append(is_max)
        found = found + is_max

    bq = bqkv_ref[...]
    qkv = jnp.zeros((BLK, 3 * DIM), jnp.float32)
    for e in range(E):
        qkv = qkv + masks[e] * (
            jnp.dot(x, wqkv_ref[e], preferred_element_type=jnp.float32)
            + bq[e:e + 1, :])

    qk = qkv[:, :2 * DIM]
    qk = qk * cos_ref[...] + _rot_half_qk(qk) * sin_ref[...]
    qkv_s[...] = jnp.concatenate([qk, qkv[:, 2 * DIM:]], axis=1
                                 ).reshape(WIN, 64, 3 * DIM)

    for w in range(NWIN):
        win = qkv_s[:, w * WIN:(w + 1) * WIN, :].reshape(N, 3 * DIM)
        outs = []
        for h in range(HEADS):
            qh = win[:, h * HD:(h + 1) * HD] * SCALE
            kh = win[:, DIM + h * HD:DIM + (h + 1) * HD]
            vh = win[:, 2 * DIM + h * HD:2 * DIM + (h + 1) * HD]
            s = lax.dot_general(qh, kh, (((1,), (1,)), ((), ())),
                                preferred_element_type=jnp.float32)
            s = s - jnp.max(s, axis=-1, keepdims=True)
            p = jnp.exp(s)
            p = p / jnp.sum(p, axis=-1, keepdims=True)
            outs.append(jnp.dot(p, vh, preferred_element_type=jnp.float32))
        att_s[:, w * WIN:(w + 1) * WIN, :] = jnp.concatenate(
            outs, axis=1).reshape(WIN, WIN, DIM)

    o = att_s[...].reshape(BLK, DIM)
    bp = bproj_ref[...]
    acc = jnp.zeros((BLK, DIM), jnp.float32)
    for e in range(E):
        acc = acc + masks[e] * (
            jnp.dot(o, wproj_ref[e], preferred_element_type=jnp.float32)
            + bp[e:e + 1, :])
    out_ref[...] = acc * pmax


@jax.jit
def kernel(x, Wqkv, bqkv, Wproj, bproj, Wr1, br1, Wr2, br2):
    Bs, H, W, C = x.shape
    xf = x.reshape(-1, C)
    T = xf.shape[0]
    grid = T // BLK
    out = pl.pallas_call(
        _body,
        grid=(grid,),
        in_specs=[
            pl.BlockSpec((BLK, DIM), lambda i: (i, 0)),
            pl.BlockSpec((E, DIM, 3 * DIM), lambda i: (0, 0, 0)),
            pl.BlockSpec((E, 3 * DIM), lambda i: (0, 0)),
            pl.BlockSpec((E, DIM, DIM), lambda i: (0, 0, 0)),
            pl.BlockSpec((E, DIM), lambda i: (0, 0)),
            pl.BlockSpec((DIM, RHID), lambda i: (0, 0)),
            pl.BlockSpec((1, RHID), lambda i: (0, 0)),
            pl.BlockSpec((RHID, E), lambda i: (0, 0)),
            pl.BlockSpec((1, E), lambda i: (0, 0)),
            pl.BlockSpec((BLK, 2 * DIM), lambda i: (0, 0)),
            pl.BlockSpec((BLK, 2 * DIM), lambda i: (0, 0)),
        ],
        out_specs=pl.BlockSpec((BLK, DIM), lambda i: (i, 0)),
        out_shape=jax.ShapeDtypeStruct((T, DIM), jnp.float32),
        scratch_shapes=[
            pltpu.VMEM((WIN, 64, 3 * DIM), jnp.float32),
            pltpu.VMEM((WIN, 64, DIM), jnp.float32),
        ],
    )(xf, Wqkv, bqkv, Wproj, bproj, Wr1, br1.reshape(1, RHID),
      Wr2, br2.reshape(1, E), jnp.asarray(_COS_B), jnp.asarray(_SIN_B))
    return out.reshape(Bs, H, W, C)
